# baseline (speedup 1.0000x reference)
import functools

import jax
import jax.numpy as jnp
from jax import lax
from jax.experimental import pallas as pl
from jax.experimental.pallas import tpu as pltpu
from jax.experimental.pallas import tpu_sc as plsc

NC = 2
NS = 16
NW = NC * NS


def _deg_body(row_hbm, zeros_hbm, ones_hbm, out_hbm, idx_v, ones_v, tmp_v,
              deg_acc, n, per_w, c1):
    cid = lax.axis_index("c")
    sid = lax.axis_index("s")
    wid = cid * NS + sid

    @pl.when(sid == 0)
    def _():
        pltpu.sync_copy(zeros_hbm, deg_acc)

    pltpu.sync_copy(ones_hbm, ones_v)
    plsc.subcore_barrier()

    base = wid * per_w
    for g in range(per_w // c1):
        pltpu.sync_copy(row_hbm.at[pl.ds(base + g * c1, c1)], idx_v)
        pltpu.sync_copy(ones_v, deg_acc.at[idx_v], add=True)

    plsc.subcore_barrier()
    chunk = 1000
    @pl.when(sid < n // chunk)
    def _():
        pltpu.sync_copy(deg_acc.at[pl.ds(sid * chunk, chunk)], tmp_v)
        pltpu.sync_copy(tmp_v, out_hbm.at[pl.ds(cid * n + sid * chunk, chunk)])


def _deg_partials(row, n, e):
    per_w = e // NW
    c1 = 2000
    assert per_w % c1 == 0 and n % 1000 == 0 and NS >= n // 1000
    mesh = plsc.VectorSubcoreMesh(core_axis_name="c", subcore_axis_name="s",
                                  num_cores=NC, num_subcores=NS)
    body = functools.partial(_deg_body, n=n, per_w=per_w, c1=c1)
    f = pl.kernel(
        body,
        out_type=jax.ShapeDtypeStruct((NC * n,), jnp.float32),
        mesh=mesh,
        scratch_types=[
            pltpu.VMEM((c1,), jnp.int32),
            pltpu.VMEM((c1,), jnp.float32),
            pltpu.VMEM((1000,), jnp.float32),
            pltpu.VMEM_SHARED((n,), jnp.float32),
        ],
        name="sc_degree_histogram",
    )
    zeros_n = jnp.zeros((n,), jnp.float32)
    ones_c = jnp.ones((c1,), jnp.float32)
    return f(row, zeros_n, ones_c)


NBUF = 6
NIDX = NBUF + 1


def _agg_body(u_hbm, row_hbm, col_hbm, zeros_hbm, out_hbm, *scr,
              n, d, per_w, c):
    rowch = scr[0:NIDX]
    colch = scr[NIDX:2 * NIDX]
    rows = scr[2 * NIDX:2 * NIDX + NBUF]
    o = 2 * NIDX + NBUF
    sems = scr[o:o + NBUF]
    ssems = scr[o + NBUF:o + 2 * NBUF]
    rsems = scr[o + 2 * NBUF:o + 2 * NBUF + NIDX]
    csems = scr[o + 2 * NBUF + NIDX:o + 2 * NBUF + 2 * NIDX]
    acc = scr[o + 2 * NBUF + 2 * NIDX]
    cid = lax.axis_index("c")
    sid = lax.axis_index("s")
    wid = cid * NS + sid
    nch = per_w // c
    rows0 = rows[0]

    rpt = 1000
    nzt = n // rpt

    @pl.when(sid < nzt)
    def _():
        pltpu.sync_copy(zeros_hbm.at[pl.ds(sid * rpt, rpt)],
                        acc.at[pl.ds(sid * rpt, rpt)])

    base = wid * per_w
    plsc.subcore_barrier()

    descs = [None] * NBUF
    sdescs = [None] * NBUF
    rdescs = [None] * NIDX
    cdescs = [None] * NIDX

    def issue_idx(h):
        hb = h % NIDX
        off = base + h * c
        rdescs[hb] = pltpu.async_copy(row_hbm.at[pl.ds(off, c)],
                                      rowch[hb], rsems[hb])
        cdescs[hb] = pltpu.async_copy(col_hbm.at[pl.ds(off, c)],
                                      colch[hb], csems[hb])

    def issue_gather(h):
        hb = h % NIDX
        cdescs[hb].wait()
        descs[h % NBUF] = pltpu.async_copy(u_hbm.at[colch[hb]],
                                           rows[h % NBUF], sems[h % NBUF])

    for g in range(min(NBUF, nch)):
        issue_idx(g)
    for g in range(min(NBUF - 1, nch)):
        issue_gather(g)
    for g in range(nch):
        b = g % NBUF
        gi = g % NIDX
        descs[b].wait()
        rdescs[gi].wait()
        sdescs[b] = pltpu.async_copy(rows[b], acc.at[rowch[gi]], ssems[b],
                                     add=True)
        h = g + NBUF - 1
        if h < nch:
            if g >= 1:
                sdescs[(g - 1) % NBUF].wait()
            issue_gather(h)
            h2 = g + NBUF
            if h2 < nch:
                issue_idx(h2)
    for g in range(max(0, nch - NBUF), nch):
        sdescs[g % NBUF].wait()

    plsc.subcore_barrier()
    @pl.when(sid < nzt)
    def _():
        done = 0
        while done < rpt:
            m = min(c, rpt - done)
            pltpu.sync_copy(acc.at[pl.ds(sid * rpt + done, m)],
                            rows0.at[pl.ds(0, m)])
            pltpu.sync_copy(rows0.at[pl.ds(0, m)],
                            out_hbm.at[pl.ds(cid * n + sid * rpt + done, m)])
            done += m


def _aggregate_partials(u, row, col, n, d, e):
    per_w = e // NW
    c = 40
    assert per_w % c == 0 and n % 1000 == 0
    mesh = plsc.VectorSubcoreMesh(core_axis_name="c", subcore_axis_name="s",
                                  num_cores=NC, num_subcores=NS)
    body = functools.partial(_agg_body, n=n, d=d, per_w=per_w, c=c)
    f = pl.kernel(
        body,
        out_type=jax.ShapeDtypeStruct((NC * n, d), jnp.float32),
        mesh=mesh,
        scratch_types=(
            [pltpu.VMEM((c,), jnp.int32) for _ in range(2 * NIDX)]
            + [pltpu.VMEM((c, d), jnp.float32) for _ in range(NBUF)]
            + [pltpu.SemaphoreType.DMA for _ in range(2 * NBUF + 2 * NIDX)]
            + [pltpu.VMEM_SHARED((n, d), jnp.float32)]
        ),
        name="sc_edge_aggregate",
    )
    zeros_nd = jnp.zeros((n, d), jnp.float32)
    return f(u, row, col, zeros_nd)


def _pre_body(x_ref, w_ref, b_ref, deg2_ref, u_ref):
    xl = lax.dot_general(x_ref[...], w_ref[...],
                         (((1,), (1,)), ((), ())),
                         preferred_element_type=jnp.float32) + b_ref[...]
    deg = jnp.sum(deg2_ref[...], axis=1, keepdims=True) + 1.0
    dinv = lax.rsqrt(deg)
    u_ref[...] = dinv * xl


def _tc_pre(x, w_lin, b_lin, deg2, n, d):
    blk = 2000
    grid = n // blk
    return pl.pallas_call(
        _pre_body,
        grid=(grid,),
        in_specs=[
            pl.BlockSpec((blk, d), lambda i: (i, 0)),
            pl.BlockSpec((d, d), lambda i: (0, 0)),
            pl.BlockSpec((1, d), lambda i: (0, 0)),
            pl.BlockSpec((blk, 2), lambda i: (i, 0)),
        ],
        out_specs=pl.BlockSpec((blk, d), lambda i: (i, 0)),
        out_shape=jax.ShapeDtypeStruct((n, d), jnp.float32),
        name="tc_lin_scale",
    )(x, w_lin, b_lin.reshape(1, d), deg2)


def _post_body(s0_ref, s1_ref, u_ref, deg2_ref, w1_ref, b1_ref, w2_ref,
               b2_ref, out_ref):
    deg = jnp.sum(deg2_ref[...], axis=1, keepdims=True) + 1.0
    dinv = lax.rsqrt(deg)
    r = dinv * (s0_ref[...] + s1_ref[...] + u_ref[...])
    z = jnp.maximum(r, 0.0)
    h = lax.dot_general(z, w1_ref[...], (((1,), (1,)), ((), ())),
                        preferred_element_type=jnp.float32) + b1_ref[...]
    h = jnp.maximum(h, 0.0)
    out_ref[...] = lax.dot_general(h, w2_ref[...], (((1,), (1,)), ((), ())),
                                   preferred_element_type=jnp.float32) + b2_ref[...]


def _tc_post(s_all, u, deg2, w1, b1, w2, b2, n, d):
    blk = 2000
    grid = n // blk
    nb = n // blk
    return pl.pallas_call(
        _post_body,
        grid=(grid,),
        in_specs=[
            pl.BlockSpec((blk, d), lambda i: (i, 0)),
            pl.BlockSpec((blk, d), lambda i, _nb=nb: (i + _nb, 0)),
            pl.BlockSpec((blk, d), lambda i: (i, 0)),
            pl.BlockSpec((blk, 2), lambda i: (i, 0)),
            pl.BlockSpec((d, d), lambda i: (0, 0)),
            pl.BlockSpec((1, d), lambda i: (0, 0)),
            pl.BlockSpec((d, d), lambda i: (0, 0)),
            pl.BlockSpec((1, d), lambda i: (0, 0)),
        ],
        out_specs=pl.BlockSpec((blk, d), lambda i: (i, 0)),
        out_shape=jax.ShapeDtypeStruct((n, d), jnp.float32),
        name="tc_norm_mlp",
    )(s_all, s_all, u, deg2, w1, b1.reshape(1, d), w2, b2.reshape(1, d))


def kernel(x, edge_index, W_lin, b_lin, W1, b1, W2, b2):
    n, d = x.shape
    e = edge_index.shape[1]
    row = edge_index[0]
    col = edge_index[1]

    deg_flat = _deg_partials(row, n, e)
    deg2 = deg_flat.reshape(NC, n).T
    u = _tc_pre(x, W_lin, b_lin, deg2, n, d)
    s_all = _aggregate_partials(u, row, col, n, d, e)
    return _tc_post(s_all, u, deg2, W1, b1, W2, b2, n, d)


# speedup vs baseline: 1.3911x; 1.3911x over previous
import functools

import jax
import jax.numpy as jnp
from jax import lax
from jax.experimental import pallas as pl
from jax.experimental.pallas import tpu as pltpu
from jax.experimental.pallas import tpu_sc as plsc

NC = 2
NS = 16
NW = NC * NS


def _split_body(ei_ref, row_ref, col_ref):
    v = ei_ref[...]
    row_ref[...] = v[0, :]
    col_ref[...] = v[1, :]


def _split_edges(edge_index, e):
    return pl.pallas_call(
        _split_body,
        out_shape=[jax.ShapeDtypeStruct((e,), jnp.int32),
                   jax.ShapeDtypeStruct((e,), jnp.int32)],
        name="tc_edge_split",
    )(edge_index)


NB1 = 3


def _deg_body(row_hbm, zeros_hbm, ones_hbm, out_hbm, *scr, n, per_w, c1):
    idxs = scr[0:NB1]
    ones_v = scr[NB1]
    tmp_v = scr[NB1 + 1]
    isems = scr[NB1 + 2:2 * NB1 + 2]
    ssems = scr[2 * NB1 + 2:3 * NB1 + 2]
    deg_acc = scr[3 * NB1 + 2]
    cid = lax.axis_index("c")
    sid = lax.axis_index("s")
    wid = cid * NS + sid
    nch = per_w // c1

    @pl.when(sid == 0)
    def _():
        pltpu.sync_copy(zeros_hbm, deg_acc)

    pltpu.sync_copy(ones_hbm, ones_v)
    plsc.subcore_barrier()

    base = wid * per_w
    descs = [None] * NB1
    sdescs = [None] * NB1

    def issue_idx(h):
        hb = h % NB1
        descs[hb] = pltpu.async_copy(row_hbm.at[pl.ds(base + h * c1, c1)],
                                     idxs[hb], isems[hb])

    for g in range(min(NB1 - 1, nch)):
        issue_idx(g)
    for g in range(nch):
        b = g % NB1
        descs[b].wait()
        sdescs[b] = pltpu.async_copy(ones_v, deg_acc.at[idxs[b]], ssems[b],
                                     add=True)
        h = g + NB1 - 1
        if h < nch:
            if g >= 1:
                sdescs[(g - 1) % NB1].wait()
            issue_idx(h)
    for g in range(max(0, nch - NB1), nch):
        sdescs[g % NB1].wait()

    plsc.subcore_barrier()
    chunk = 1000
    @pl.when(sid < n // chunk)
    def _():
        pltpu.sync_copy(deg_acc.at[pl.ds(sid * chunk, chunk)], tmp_v)
        pltpu.sync_copy(tmp_v, out_hbm.at[pl.ds(cid * n + sid * chunk, chunk)])


def _deg_partials(row, n, e):
    per_w = e // NW
    c1 = 2000
    assert per_w % c1 == 0 and n % 1000 == 0 and NS >= n // 1000
    mesh = plsc.VectorSubcoreMesh(core_axis_name="c", subcore_axis_name="s",
                                  num_cores=NC, num_subcores=NS)
    body = functools.partial(_deg_body, n=n, per_w=per_w, c1=c1)
    f = pl.kernel(
        body,
        out_type=jax.ShapeDtypeStruct((NC * n,), jnp.float32),
        mesh=mesh,
        scratch_types=(
            [pltpu.VMEM((c1,), jnp.int32) for _ in range(NB1)]
            + [pltpu.VMEM((c1,), jnp.float32),
               pltpu.VMEM((1000,), jnp.float32)]
            + [pltpu.SemaphoreType.DMA for _ in range(2 * NB1)]
            + [pltpu.VMEM_SHARED((n,), jnp.float32)]
        ),
        name="sc_degree_histogram",
    )
    zeros_n = jnp.zeros((n,), jnp.float32)
    ones_c = jnp.ones((c1,), jnp.float32)
    return f(row, zeros_n, ones_c)


NBUF = 4
NIDX = NBUF + 1


def _agg_body(u_hbm, row_hbm, col_hbm, zeros_hbm, out_hbm, *scr,
              n, d, per_w, c):
    rowch = scr[0:NIDX]
    colch = scr[NIDX:2 * NIDX]
    rows = scr[2 * NIDX:2 * NIDX + NBUF]
    o = 2 * NIDX + NBUF
    sems = scr[o:o + NBUF]
    ssems = scr[o + NBUF:o + 2 * NBUF]
    rsems = scr[o + 2 * NBUF:o + 2 * NBUF + NIDX]
    csems = scr[o + 2 * NBUF + NIDX:o + 2 * NBUF + 2 * NIDX]
    acc = scr[o + 2 * NBUF + 2 * NIDX]
    cid = lax.axis_index("c")
    sid = lax.axis_index("s")
    wid = cid * NS + sid
    nch = per_w // c
    rows0 = rows[0]

    rpt = 1000
    nzt = n // rpt

    @pl.when(sid < nzt)
    def _():
        pltpu.sync_copy(zeros_hbm.at[pl.ds(sid * rpt, rpt)],
                        acc.at[pl.ds(sid * rpt, rpt)])

    base = wid * per_w
    plsc.subcore_barrier()

    descs = [None] * NBUF
    sdescs = [None] * NBUF
    rdescs = [None] * NIDX
    cdescs = [None] * NIDX

    def issue_idx(h):
        hb = h % NIDX
        off = base + h * c
        rdescs[hb] = pltpu.async_copy(row_hbm.at[pl.ds(off, c)],
                                      rowch[hb], rsems[hb])
        cdescs[hb] = pltpu.async_copy(col_hbm.at[pl.ds(off, c)],
                                      colch[hb], csems[hb])

    def issue_gather(h):
        hb = h % NIDX
        cdescs[hb].wait()
        descs[h % NBUF] = pltpu.async_copy(u_hbm.at[colch[hb]],
                                           rows[h % NBUF], sems[h % NBUF])

    for g in range(min(NBUF, nch)):
        issue_idx(g)
    for g in range(min(NBUF - 1, nch)):
        issue_gather(g)
    for g in range(nch):
        b = g % NBUF
        gi = g % NIDX
        descs[b].wait()
        rdescs[gi].wait()
        sdescs[b] = pltpu.async_copy(rows[b], acc.at[rowch[gi]], ssems[b],
                                     add=True)
        h = g + NBUF - 1
        if h < nch:
            if g >= 1:
                sdescs[(g - 1) % NBUF].wait()
            issue_gather(h)
            h2 = g + NBUF
            if h2 < nch:
                issue_idx(h2)
    for g in range(max(0, nch - NBUF), nch):
        sdescs[g % NBUF].wait()

    plsc.subcore_barrier()
    @pl.when(sid < nzt)
    def _():
        done = 0
        while done < rpt:
            m = min(c, rpt - done)
            pltpu.sync_copy(acc.at[pl.ds(sid * rpt + done, m)],
                            rows0.at[pl.ds(0, m)])
            pltpu.sync_copy(rows0.at[pl.ds(0, m)],
                            out_hbm.at[pl.ds(cid * n + sid * rpt + done, m)])
            done += m


def _aggregate_partials(u, row, col, n, d, e):
    per_w = e // NW
    c = 80
    assert per_w % c == 0 and n % 1000 == 0
    mesh = plsc.VectorSubcoreMesh(core_axis_name="c", subcore_axis_name="s",
                                  num_cores=NC, num_subcores=NS)
    body = functools.partial(_agg_body, n=n, d=d, per_w=per_w, c=c)
    f = pl.kernel(
        body,
        out_type=jax.ShapeDtypeStruct((NC * n, d), jnp.float32),
        mesh=mesh,
        scratch_types=(
            [pltpu.VMEM((c,), jnp.int32) for _ in range(2 * NIDX)]
            + [pltpu.VMEM((c, d), jnp.float32) for _ in range(NBUF)]
            + [pltpu.SemaphoreType.DMA for _ in range(2 * NBUF + 2 * NIDX)]
            + [pltpu.VMEM_SHARED((n, d), jnp.float32)]
        ),
        name="sc_edge_aggregate",
    )
    zeros_nd = jnp.zeros((n, d), jnp.float32)
    return f(u, row, col, zeros_nd)


def _matmul_body(x_ref, w_ref, b_ref, xl_ref):
    xl_ref[...] = lax.dot_general(x_ref[...], w_ref[...],
                                  (((1,), (1,)), ((), ())),
                                  preferred_element_type=jnp.float32) + b_ref[...]


def _tc_matmul(x, w_lin, b_lin, n, d):
    return pl.pallas_call(
        _matmul_body,
        out_shape=jax.ShapeDtypeStruct((n, d), jnp.float32),
        name="tc_linear",
    )(x, w_lin, b_lin.reshape(1, d))


def _dinv_col(deg_ref, n):
    deg = deg_ref[pl.ds(0, n)] + deg_ref[pl.ds(n, n)] + 1.0
    return lax.rsqrt(deg)[:, None]


def _scale_body(xl_ref, deg_ref, u_ref):
    n = xl_ref.shape[0]
    u_ref[...] = _dinv_col(deg_ref, n) * xl_ref[...]


def _tc_scale(xl, deg_flat, n, d):
    return pl.pallas_call(
        _scale_body,
        out_shape=jax.ShapeDtypeStruct((n, d), jnp.float32),
        name="tc_scale",
    )(xl, deg_flat)


def _post_body(s_ref, u_ref, deg_ref, w1_ref, b1_ref, w2_ref, b2_ref,
               out_ref):
    n = u_ref.shape[0]
    r = _dinv_col(deg_ref, n) * (s_ref[pl.ds(0, n), :] + s_ref[pl.ds(n, n), :]
                                 + u_ref[...])
    z = jnp.maximum(r, 0.0)
    h = lax.dot_general(z, w1_ref[...], (((1,), (1,)), ((), ())),
                        preferred_element_type=jnp.float32) + b1_ref[...]
    h = jnp.maximum(h, 0.0)
    out_ref[...] = lax.dot_general(h, w2_ref[...], (((1,), (1,)), ((), ())),
                                   preferred_element_type=jnp.float32) + b2_ref[...]


def _tc_post(s_all, u, deg_flat, w1, b1, w2, b2, n, d):
    return pl.pallas_call(
        _post_body,
        out_shape=jax.ShapeDtypeStruct((n, d), jnp.float32),
        name="tc_norm_mlp",
    )(s_all, u, deg_flat, w1, b1.reshape(1, d), w2, b2.reshape(1, d))


def kernel(x, edge_index, W_lin, b_lin, W1, b1, W2, b2):
    n, d = x.shape
    e = edge_index.shape[1]

    row, col = _split_edges(edge_index, e)
    xl = _tc_matmul(x, W_lin, b_lin, n, d)
    deg_flat = _deg_partials(row, n, e)
    u = _tc_scale(xl, deg_flat, n, d)
    s_all = _aggregate_partials(u, row, col, n, d, e)
    return _tc_post(s_all, u, deg_flat, W1, b1, W2, b2, n, d)


# speedup vs baseline: 1.4664x; 1.0542x over previous
import functools

import jax
import jax.numpy as jnp
from jax import lax
from jax.experimental import pallas as pl
from jax.experimental.pallas import tpu as pltpu
from jax.experimental.pallas import tpu_sc as plsc

NC = 2
NS = 16
NW = NC * NS


def _split_body(ei_ref, row_ref, col_ref):
    v = ei_ref[...]
    row_ref[...] = v[0, :]
    col_ref[...] = v[1, :]


def _split_edges(edge_index, e):
    return pl.pallas_call(
        _split_body,
        out_shape=[jax.ShapeDtypeStruct((e,), jnp.int32),
                   jax.ShapeDtypeStruct((e,), jnp.int32)],
        name="tc_edge_split",
    )(edge_index)


NB1 = 3


def _deg_body(row_hbm, zeros_hbm, ones_hbm, out_hbm, *scr, n, per_w, c1):
    idxs = scr[0:NB1]
    ones_v = scr[NB1]
    tmp_v = scr[NB1 + 1]
    isems = scr[NB1 + 2:2 * NB1 + 2]
    ssems = scr[2 * NB1 + 2:3 * NB1 + 2]
    deg_acc = scr[3 * NB1 + 2]
    cid = lax.axis_index("c")
    sid = lax.axis_index("s")
    wid = cid * NS + sid
    nch = per_w // c1

    @pl.when(sid == 0)
    def _():
        pltpu.sync_copy(zeros_hbm, deg_acc)

    pltpu.sync_copy(ones_hbm, ones_v)
    plsc.subcore_barrier()

    base = wid * per_w
    descs = [None] * NB1
    sdescs = [None] * NB1

    def issue_idx(h):
        hb = h % NB1
        descs[hb] = pltpu.async_copy(row_hbm.at[pl.ds(base + h * c1, c1)],
                                     idxs[hb], isems[hb])

    for g in range(min(NB1 - 1, nch)):
        issue_idx(g)
    for g in range(nch):
        b = g % NB1
        descs[b].wait()
        sdescs[b] = pltpu.async_copy(ones_v, deg_acc.at[idxs[b]], ssems[b],
                                     add=True)
        h = g + NB1 - 1
        if h < nch:
            if g >= 1:
                sdescs[(g - 1) % NB1].wait()
            issue_idx(h)
    for g in range(max(0, nch - NB1), nch):
        sdescs[g % NB1].wait()

    plsc.subcore_barrier()
    chunk = 1000
    @pl.when(sid < n // chunk)
    def _():
        pltpu.sync_copy(deg_acc.at[pl.ds(sid * chunk, chunk)], tmp_v)
        pltpu.sync_copy(tmp_v, out_hbm.at[pl.ds(cid * n + sid * chunk, chunk)])


def _deg_partials(row, n, e):
    per_w = e // NW
    c1 = 2000
    assert per_w % c1 == 0 and n % 1000 == 0 and NS >= n // 1000
    mesh = plsc.VectorSubcoreMesh(core_axis_name="c", subcore_axis_name="s",
                                  num_cores=NC, num_subcores=NS)
    body = functools.partial(_deg_body, n=n, per_w=per_w, c1=c1)
    f = pl.kernel(
        body,
        out_type=jax.ShapeDtypeStruct((NC * n,), jnp.float32),
        mesh=mesh,
        scratch_types=(
            [pltpu.VMEM((c1,), jnp.int32) for _ in range(NB1)]
            + [pltpu.VMEM((c1,), jnp.float32),
               pltpu.VMEM((1000,), jnp.float32)]
            + [pltpu.SemaphoreType.DMA for _ in range(2 * NB1)]
            + [pltpu.VMEM_SHARED((n,), jnp.float32)]
        ),
        name="sc_degree_histogram",
    )
    zeros_n = jnp.zeros((n,), jnp.float32)
    ones_c = jnp.ones((c1,), jnp.float32)
    return f(row, zeros_n, ones_c)


NBUF = 4
NIDX = NBUF + 1


def _agg_body(u_hbm, row_hbm, col_hbm, zeros_hbm, out_hbm, *scr,
              n, d, per_w, c):
    rowch = scr[0:NIDX]
    colch = scr[NIDX:2 * NIDX]
    rows = scr[2 * NIDX:2 * NIDX + NBUF]
    o = 2 * NIDX + NBUF
    sems = scr[o:o + NBUF]
    ssems = scr[o + NBUF:o + 2 * NBUF]
    rsems = scr[o + 2 * NBUF:o + 2 * NBUF + NIDX]
    csems = scr[o + 2 * NBUF + NIDX:o + 2 * NBUF + 2 * NIDX]
    acc = scr[o + 2 * NBUF + 2 * NIDX]
    cid = lax.axis_index("c")
    sid = lax.axis_index("s")
    wid = cid * NS + sid
    nch = per_w // c
    rows0 = rows[0]

    rpt = 1000
    nzt = n // rpt
    zdesc = [None]

    zbase = pl.multiple_of(sid * rpt, 8)

    @pl.when(sid < nzt)
    def _():
        zdesc[0] = pltpu.async_copy(zeros_hbm.at[pl.ds(zbase, rpt)],
                                    acc.at[pl.ds(zbase, rpt)], ssems[0])

    base = wid * per_w

    descs = [None] * NBUF
    sdescs = [None] * NBUF
    rdescs = [None] * NIDX
    cdescs = [None] * NIDX

    def issue_idx(h):
        hb = h % NIDX
        off = base + h * c
        rdescs[hb] = pltpu.async_copy(row_hbm.at[pl.ds(off, c)],
                                      rowch[hb], rsems[hb])
        cdescs[hb] = pltpu.async_copy(col_hbm.at[pl.ds(off, c)],
                                      colch[hb], csems[hb])

    def issue_gather(h):
        hb = h % NIDX
        cdescs[hb].wait()
        descs[h % NBUF] = pltpu.async_copy(u_hbm.at[colch[hb]],
                                           rows[h % NBUF], sems[h % NBUF])

    for g in range(min(NBUF, nch)):
        issue_idx(g)
    for g in range(min(NBUF - 1, nch)):
        issue_gather(g)

    @pl.when(sid < nzt)
    def _():
        zdesc[0].wait()

    plsc.subcore_barrier()
    for g in range(nch):
        b = g % NBUF
        gi = g % NIDX
        descs[b].wait()
        rdescs[gi].wait()
        sdescs[b] = pltpu.async_copy(rows[b], acc.at[rowch[gi]], ssems[b],
                                     add=True)
        h = g + NBUF - 1
        if h < nch:
            if g >= 1:
                sdescs[(g - 1) % NBUF].wait()
            issue_gather(h)
            h2 = g + NBUF
            if h2 < nch:
                issue_idx(h2)
    for g in range(max(0, nch - NBUF), nch):
        sdescs[g % NBUF].wait()

    plsc.subcore_barrier()
    @pl.when(sid < nzt)
    def _():
        pieces = []
        done = 0
        while done < rpt:
            m = min(c, rpt - done)
            pieces.append((done, m))
            done += m
        ind = [None] * NBUF
        outd = [None] * NBUF
        for k in range(len(pieces) + 1):
            if k < len(pieces):
                b = k % NBUF
                if outd[b] is not None:
                    outd[b].wait()
                off, m = pieces[k]
                ind[b] = pltpu.async_copy(
                    acc.at[pl.ds(pl.multiple_of(zbase + off, 8), m)],
                    rows[b].at[pl.ds(0, m)], sems[b])
            if k >= 1:
                kb = (k - 1) % NBUF
                off, m = pieces[k - 1]
                ind[kb].wait()
                outd[kb] = pltpu.async_copy(
                    rows[kb].at[pl.ds(0, m)],
                    out_hbm.at[pl.ds(pl.multiple_of(cid * n + zbase + off, 8),
                                     m)],
                    ssems[kb])
        for b in range(NBUF):
            if outd[b] is not None:
                outd[b].wait()


def _aggregate_partials(u, row, col, n, d, e):
    per_w = e // NW
    c = 80
    assert per_w % c == 0 and n % 1000 == 0
    mesh = plsc.VectorSubcoreMesh(core_axis_name="c", subcore_axis_name="s",
                                  num_cores=NC, num_subcores=NS)
    body = functools.partial(_agg_body, n=n, d=d, per_w=per_w, c=c)
    f = pl.kernel(
        body,
        out_type=jax.ShapeDtypeStruct((NC * n, d), jnp.float32),
        mesh=mesh,
        scratch_types=(
            [pltpu.VMEM((c,), jnp.int32) for _ in range(2 * NIDX)]
            + [pltpu.VMEM((c, d), jnp.float32) for _ in range(NBUF)]
            + [pltpu.SemaphoreType.DMA for _ in range(2 * NBUF + 2 * NIDX)]
            + [pltpu.VMEM_SHARED((n, d), jnp.float32)]
        ),
        name="sc_edge_aggregate",
    )
    zeros_nd = jnp.zeros((n, d), jnp.float32)
    return f(u, row, col, zeros_nd)


def _matmul_body(x_ref, w_ref, b_ref, xl_ref):
    xl_ref[...] = lax.dot_general(x_ref[...], w_ref[...],
                                  (((1,), (1,)), ((), ())),
                                  preferred_element_type=jnp.float32) + b_ref[...]


def _tc_matmul(x, w_lin, b_lin, n, d):
    return pl.pallas_call(
        _matmul_body,
        out_shape=jax.ShapeDtypeStruct((n, d), jnp.float32),
        name="tc_linear",
    )(x, w_lin, b_lin.reshape(1, d))


def _dinv_col(deg_ref, n):
    deg = deg_ref[pl.ds(0, n)] + deg_ref[pl.ds(n, n)] + 1.0
    return lax.rsqrt(deg)[:, None]


def _scale_body(xl_ref, deg_ref, u_ref):
    n = xl_ref.shape[0]
    u_ref[...] = _dinv_col(deg_ref, n) * xl_ref[...]


def _tc_scale(xl, deg_flat, n, d):
    return pl.pallas_call(
        _scale_body,
        out_shape=jax.ShapeDtypeStruct((n, d), jnp.float32),
        name="tc_scale",
    )(xl, deg_flat)


def _post_body(s_ref, u_ref, deg_ref, w1_ref, b1_ref, w2_ref, b2_ref,
               out_ref):
    n = u_ref.shape[0]
    r = _dinv_col(deg_ref, n) * (s_ref[pl.ds(0, n), :] + s_ref[pl.ds(n, n), :]
                                 + u_ref[...])
    z = jnp.maximum(r, 0.0)
    h = lax.dot_general(z, w1_ref[...], (((1,), (1,)), ((), ())),
                        preferred_element_type=jnp.float32) + b1_ref[...]
    h = jnp.maximum(h, 0.0)
    out_ref[...] = lax.dot_general(h, w2_ref[...], (((1,), (1,)), ((), ())),
                                   preferred_element_type=jnp.float32) + b2_ref[...]


def _tc_post(s_all, u, deg_flat, w1, b1, w2, b2, n, d):
    return pl.pallas_call(
        _post_body,
        out_shape=jax.ShapeDtypeStruct((n, d), jnp.float32),
        name="tc_norm_mlp",
    )(s_all, u, deg_flat, w1, b1.reshape(1, d), w2, b2.reshape(1, d))


def kernel(x, edge_index, W_lin, b_lin, W1, b1, W2, b2):
    n, d = x.shape
    e = edge_index.shape[1]

    row, col = _split_edges(edge_index, e)
    xl = _tc_matmul(x, W_lin, b_lin, n, d)
    deg_flat = _deg_partials(row, n, e)
    u = _tc_scale(xl, deg_flat, n, d)
    s_all = _aggregate_partials(u, row, col, n, d, e)
    return _tc_post(s_all, u, deg_flat, W1, b1, W2, b2, n, d)
